# baseline (speedup 1.0000x reference)
import jax
import jax.numpy as jnp
from jax import lax
from jax.experimental import pallas as pl
from jax.experimental.pallas import tpu as pltpu
from jax.experimental.pallas import tpu_sc as plsc

M = 1000000
D = 59
DP = 64
B = 262144

NC = 2
NT = 16
L = 16

NSEG = 50
SEG = M // NSEG
NPASS = NSEG // NC
SEGU = SEG * D // L
UPT = 4609
UREM = SEGU - NT * UPT
DUMPU = SEGU

IDXSL = B // NT
CHUNKS = IDXSL // L

FL = 128
NU = 5
BUF = FL + L
TRASH = BUF
BUFA = BUF + L


def _body(mem16, valp, idx, out16, seg, idxv, abuf, obuf, jbuf, jfl, aidx,
          valbuf, pieces, sem):
    c = lax.axis_index("c")
    t = lax.axis_index("s")
    iota = lax.iota(jnp.int32, L)

    apad = DUMPU + iota
    opad = jnp.zeros((L,), jnp.int32)
    jpad = iota

    pltpu.sync_copy(idx.at[pl.ds(t * IDXSL, IDXSL)], idxv)

    def reset_bufs():
        for k in range(BUF // L):
            abuf[pl.ds(k * L, L)] = apad
            obuf[pl.ds(k * L, L)] = opad
            jbuf[pl.ds(k * L, L)] = jpad

    def pad_at(pos):
        plsc.store_scatter(abuf, [pos + iota], apad)
        plsc.store_scatter(obuf, [pos + iota], opad)
        plsc.store_scatter(jbuf, [pos + iota], jpad)

    def flush():
        for k in range(FL // L):
            g = pl.ds(k * L, L)
            a16 = abuf[g]
            jfl[g] = jbuf[g]
            for m in range(NU):
                aidx[m, g] = a16 + m
        pltpu.async_copy(valp.at[jfl], valbuf, sem).wait()
        orv = [obuf[pl.ds(g * L, L)] for g in range(FL // L)]
        uv = [iota + g * L for g in range(FL // L)]

        def build(p, _):
            d0 = jnp.broadcast_to(p >> 4, (L,))
            d2 = jnp.broadcast_to(p & 15, (L,))
            for g in range(FL // L):
                xs = p - orv[g]
                xsafe = jnp.minimum(xs & 127, 63)
                x = plsc.load_gather(valbuf, [uv[g], xsafe])
                plsc.store_scatter(pieces, [d0, uv[g], d2], x)
            return 0

        lax.fori_loop(0, NU * L, build, 0)
        for m in range(NU):
            pltpu.sync_copy(pieces.at[m], seg.at[aidx.at[m]], add=True)

    def pass_body(p, _):
        base = (c * NPASS + p) * SEG
        ubase = (c * NPASS + p) * SEGU

        pltpu.sync_copy(mem16.at[pl.ds(ubase + t * UPT, UPT)],
                        seg.at[pl.ds(t * UPT, UPT)])

        @pl.when(t == 0)
        def _():
            pltpu.sync_copy(mem16.at[pl.ds(ubase + NT * UPT, UREM)],
                            seg.at[pl.ds(NT * UPT, UREM)])

        plsc.subcore_barrier()
        reset_bufs()

        def scan_body(i, fill):
            v = idxv[pl.ds(i * L, L)]
            in_seg = (v >= base) & (v < base + SEG)
            nmatch = plsc.all_reduce_population_count(in_seg)[0]

            @pl.when(nmatch > 0)
            def _():
                w0 = (v - base) * D
                jvec = iota + (t * IDXSL + i * L)
                inc = in_seg.astype(jnp.int32)
                excl = plsc.cumsum(inc) - inc
                dest = jnp.where(in_seg, fill + excl, TRASH + iota)
                plsc.store_scatter(abuf, [dest], w0 >> 4)
                plsc.store_scatter(obuf, [dest], w0 & 15)
                plsc.store_scatter(jbuf, [dest], jvec)

            fill = fill + nmatch
            full = fill >= FL

            @pl.when(full)
            def _():
                flush()
                nf = fill - FL
                lv_a = abuf[pl.ds(FL, L)]
                lv_o = obuf[pl.ds(FL, L)]
                lv_j = jbuf[pl.ds(FL, L)]
                reset_bufs()
                keep = iota < nf
                kdest = jnp.where(keep, iota, TRASH + iota)
                plsc.store_scatter(abuf, [kdest], lv_a)
                plsc.store_scatter(obuf, [kdest], lv_o)
                plsc.store_scatter(jbuf, [kdest], lv_j)
                pad_at(nf)

            return jnp.where(full, fill - FL, fill)

        fill = lax.fori_loop(0, CHUNKS, scan_body, jnp.int32(0))
        pad_at(fill)

        @pl.when(fill > 0)
        def _():
            flush()

        plsc.subcore_barrier()
        pltpu.sync_copy(seg.at[pl.ds(t * UPT, UPT)],
                        out16.at[pl.ds(ubase + t * UPT, UPT)])

        @pl.when(t == 0)
        def _():
            pltpu.sync_copy(seg.at[pl.ds(NT * UPT, UREM)],
                            out16.at[pl.ds(ubase + NT * UPT, UREM)])

        plsc.subcore_barrier()
        return 0

    lax.fori_loop(0, NPASS, pass_body, 0)


@jax.jit
def _scatter_add(mem16, valp, idx):
    mesh = plsc.VectorSubcoreMesh(
        core_axis_name="c", subcore_axis_name="s", num_cores=NC, num_subcores=NT
    )
    return pl.kernel(
        _body,
        out_type=jax.ShapeDtypeStruct((M * D // L, L), jnp.float32),
        mesh=mesh,
        compiler_params=pltpu.CompilerParams(
            needs_layout_passes=False, use_tc_tiling_on_sc=False
        ),
        scratch_types=[
            pltpu.VMEM_SHARED((SEGU + 20, L), jnp.float32),
            pltpu.VMEM((IDXSL,), jnp.int32),
            pltpu.VMEM((BUFA,), jnp.int32),
            pltpu.VMEM((BUFA,), jnp.int32),
            pltpu.VMEM((BUFA,), jnp.int32),
            pltpu.VMEM((FL,), jnp.int32),
            pltpu.VMEM((NU, FL), jnp.int32),
            pltpu.VMEM((FL, DP), jnp.float32),
            pltpu.VMEM((NU, FL, L), jnp.float32),
            pltpu.SemaphoreType.DMA,
        ],
    )(mem16, valp, idx)


def kernel(mem, val, idx):
    mem16 = jnp.reshape(mem, (M * D // L, L))
    valp = jnp.pad(val, ((0, 0), (0, DP - D)))
    out16 = _scatter_add(mem16, valp, idx)
    return jnp.reshape(out16, (M, D))


# speedup vs baseline: 1.2972x; 1.2972x over previous
import jax
import jax.numpy as jnp
from jax import lax
from jax.experimental import pallas as pl
from jax.experimental.pallas import tpu as pltpu
from jax.experimental.pallas import tpu_sc as plsc

M = 1000000
D = 59
DP = 64
B = 262144

NC = 2
NT = 16
L = 16

NSEG = 40
SEG = M // NSEG
NPASS = NSEG // NC
TPT = 1568
LAST_START = SEG - TPT

IDXSL = B // NT
CHUNKS = IDXSL // L

FL = 96
BUF = FL + L
TRASH = BUF
BUFA = BUF + L


def _body(mem, val, idx, out, seg, idxv, lbuf, jbuf, lfl, jfl, valbuf, gsem,
          ssem):
    c = lax.axis_index("c")
    t = lax.axis_index("s")
    iota = lax.iota(jnp.int32, L)

    lpad = iota + SEG
    jpad = iota

    pltpu.sync_copy(idx.at[pl.ds(t * IDXSL, IDXSL)], idxv)

    start_t = jnp.minimum(t * TPT, LAST_START)

    def reset_bufs():
        for k in range(BUF // L):
            lbuf[pl.ds(k * L, L)] = lpad
            jbuf[pl.ds(k * L, L)] = jpad

    def flush(cnt):
        s = cnt & 1
        p = 1 - s

        @pl.when(cnt >= 2)
        def _():
            pltpu.make_async_copy(valbuf.at[s], seg.at[lfl.at[s]],
                                  ssem.at[s]).wait()

        for k in range(FL // L):
            g = pl.ds(k * L, L)
            lfl.at[s][g] = lbuf[g]
            jfl.at[s][g] = jbuf[g]
        pltpu.async_copy(val.at[jfl.at[s]], valbuf.at[s], gsem.at[s])

        @pl.when(cnt >= 1)
        def _():
            pltpu.make_async_copy(val.at[jfl.at[p]], valbuf.at[p],
                                  gsem.at[p]).wait()
            pltpu.async_copy(valbuf.at[p], seg.at[lfl.at[p]], ssem.at[p],
                             add=True)

    def pass_body(p, _):
        base = (c * NPASS + p) * SEG
        row0 = base + start_t

        pltpu.sync_copy(mem.at[pl.ds(row0, TPT)], seg.at[pl.ds(start_t, TPT)])
        plsc.subcore_barrier()

        reset_bufs()

        def scan_body(i, carry):
            fill, cnt = carry
            v = idxv[pl.ds(i * L, L)]
            in_seg = (v >= base) & (v < base + SEG)
            nmatch = plsc.all_reduce_population_count(in_seg)[0]

            @pl.when(nmatch > 0)
            def _():
                lidx = v - base
                jvec = iota + (t * IDXSL + i * L)
                inc = in_seg.astype(jnp.int32)
                excl = plsc.cumsum(inc) - inc
                dest = jnp.where(in_seg, fill + excl, TRASH + iota)
                plsc.store_scatter(lbuf, [dest], lidx)
                plsc.store_scatter(jbuf, [dest], jvec)

            fill = fill + nmatch
            full = fill >= FL

            @pl.when(full)
            def _():
                flush(cnt)
                nf = fill - FL
                lv_l = lbuf[pl.ds(FL, L)]
                lv_j = jbuf[pl.ds(FL, L)]
                reset_bufs()
                keep = iota < nf
                kdest = jnp.where(keep, iota, TRASH + iota)
                plsc.store_scatter(lbuf, [kdest], lv_l)
                plsc.store_scatter(jbuf, [kdest], lv_j)
                plsc.store_scatter(lbuf, [nf + iota], lpad)
                plsc.store_scatter(jbuf, [nf + iota], jpad)

            return (jnp.where(full, fill - FL, fill),
                    cnt + full.astype(jnp.int32))

        fill, cnt = lax.fori_loop(0, CHUNKS, scan_body,
                                  (jnp.int32(0), jnp.int32(0)))

        plsc.store_scatter(lbuf, [fill + iota], lpad)
        plsc.store_scatter(jbuf, [fill + iota], jpad)

        @pl.when(fill > 0)
        def _():
            flush(cnt)

        cntf = cnt + (fill > 0).astype(jnp.int32)

        @pl.when(cntf >= 1)
        def _():
            q = (cntf - 1) & 1
            pltpu.make_async_copy(val.at[jfl.at[q]], valbuf.at[q],
                                  gsem.at[q]).wait()
            pltpu.async_copy(valbuf.at[q], seg.at[lfl.at[q]], ssem.at[q],
                             add=True)
            pltpu.make_async_copy(valbuf.at[q], seg.at[lfl.at[q]],
                                  ssem.at[q]).wait()

        @pl.when(cntf >= 2)
        def _():
            q2 = cntf & 1
            pltpu.make_async_copy(valbuf.at[q2], seg.at[lfl.at[q2]],
                                  ssem.at[q2]).wait()

        plsc.subcore_barrier()
        pltpu.sync_copy(seg.at[pl.ds(start_t, TPT)], out.at[pl.ds(row0, TPT)])
        plsc.subcore_barrier()
        return 0

    lax.fori_loop(0, NPASS, pass_body, 0)


def _pad_rows(x, blk):
    n = x.shape[0]

    def body(i_ref, o_ref):
        o_ref[...] = jnp.pad(i_ref[...], ((0, 0), (0, DP - D)))

    return pl.pallas_call(
        body,
        grid=(n // blk,),
        in_specs=[pl.BlockSpec((blk, D), lambda i: (i, 0))],
        out_specs=pl.BlockSpec((blk, DP), lambda i: (i, 0)),
        out_shape=jax.ShapeDtypeStruct((n, DP), jnp.float32),
    )(x)


def _unpad_rows(x, blk):
    n = x.shape[0]

    def body(i_ref, o_ref):
        o_ref[...] = i_ref[:, :D]

    return pl.pallas_call(
        body,
        grid=(n // blk,),
        in_specs=[pl.BlockSpec((blk, DP), lambda i: (i, 0))],
        out_specs=pl.BlockSpec((blk, D), lambda i: (i, 0)),
        out_shape=jax.ShapeDtypeStruct((n, D), jnp.float32),
    )(x)


@jax.jit
def _scatter_add(memp, valp, idx):
    mesh = plsc.VectorSubcoreMesh(
        core_axis_name="c", subcore_axis_name="s", num_cores=NC, num_subcores=NT
    )
    return pl.kernel(
        _body,
        out_type=jax.ShapeDtypeStruct((M, DP), jnp.float32),
        mesh=mesh,
        compiler_params=pltpu.CompilerParams(
            needs_layout_passes=False, use_tc_tiling_on_sc=False
        ),
        scratch_types=[
            pltpu.VMEM_SHARED((SEG + L, DP), jnp.float32),
            pltpu.VMEM((IDXSL,), jnp.int32),
            pltpu.VMEM((BUFA,), jnp.int32),
            pltpu.VMEM((BUFA,), jnp.int32),
            pltpu.VMEM((2, FL), jnp.int32),
            pltpu.VMEM((2, FL), jnp.int32),
            pltpu.VMEM((2, FL, DP), jnp.float32),
            pltpu.SemaphoreType.DMA((2,)),
            pltpu.SemaphoreType.DMA((2,)),
        ],
    )(memp, valp, idx)


def kernel(mem, val, idx):
    memp = jnp.pad(mem, ((0, 0), (0, DP - D)))
    valp = jnp.pad(val, ((0, 0), (0, DP - D)))
    outp = _scatter_add(memp, valp, idx)
    return outp[:, :D]


# speedup vs baseline: 1.2977x; 1.0004x over previous
import jax
import jax.numpy as jnp
from jax import lax
from jax.experimental import pallas as pl
from jax.experimental.pallas import tpu as pltpu
from jax.experimental.pallas import tpu_sc as plsc

M = 1000000
D = 59
DP = 64
B = 262144

NC = 2
NT = 16
L = 16

NSEG = 40
SEG = M // NSEG
NPASS = NSEG // NC
TPT = 1568
LAST_START = SEG - TPT

IDXSL = B // NT
CHUNKS = IDXSL // L

FL = 96
BUF = FL + L
TRASH = BUF
BUFA = BUF + L


def _body(mem, val, idx, out, seg, idxv, lbuf, jbuf, lfl, jfl, valbuf, gsem,
          ssem):
    c = lax.axis_index("c")
    t = lax.axis_index("s")
    iota = lax.iota(jnp.int32, L)

    lpad = iota + SEG
    jpad = iota

    pltpu.sync_copy(idx.at[pl.ds(t * IDXSL, IDXSL)], idxv)

    start_t = jnp.minimum(t * TPT, LAST_START)

    def reset_bufs():
        for k in range(BUF // L):
            lbuf[pl.ds(k * L, L)] = lpad
            jbuf[pl.ds(k * L, L)] = jpad

    def flush(cnt):
        s = cnt & 1
        p = 1 - s

        @pl.when(cnt >= 2)
        def _():
            pltpu.make_async_copy(valbuf.at[s], seg.at[lfl.at[s]],
                                  ssem.at[s]).wait()

        for k in range(FL // L):
            g = pl.ds(k * L, L)
            lfl.at[s][g] = lbuf[g]
            jfl.at[s][g] = jbuf[g]
        pltpu.async_copy(val.at[jfl.at[s]], valbuf.at[s], gsem.at[s])

        @pl.when(cnt >= 1)
        def _():
            pltpu.make_async_copy(val.at[jfl.at[p]], valbuf.at[p],
                                  gsem.at[p]).wait()
            pltpu.async_copy(valbuf.at[p], seg.at[lfl.at[p]], ssem.at[p],
                             add=True)

    def pass_body(p, _):
        base = (c * NPASS + p) * SEG
        row0 = base + start_t

        pltpu.sync_copy(mem.at[pl.ds(row0, TPT)], seg.at[pl.ds(start_t, TPT)])
        plsc.subcore_barrier()

        reset_bufs()

        def scan_body(i, carry):
            fill, cnt = carry
            v = idxv[pl.ds(i * L, L)]
            in_seg = (v >= base) & (v < base + SEG)
            nmatch = plsc.all_reduce_population_count(in_seg)[0]

            @pl.when(nmatch > 0)
            def _():
                lidx = v - base
                jvec = iota + (t * IDXSL + i * L)
                inc = in_seg.astype(jnp.int32)
                excl = plsc.cumsum(inc) - inc
                dest = jnp.where(in_seg, fill + excl, TRASH + iota)
                plsc.store_scatter(lbuf, [dest], lidx)
                plsc.store_scatter(jbuf, [dest], jvec)

            fill = fill + nmatch
            full = fill >= FL

            @pl.when(full)
            def _():
                flush(cnt)
                nf = fill - FL
                lv_l = lbuf[pl.ds(FL, L)]
                lv_j = jbuf[pl.ds(FL, L)]
                reset_bufs()
                keep = iota < nf
                kdest = jnp.where(keep, iota, TRASH + iota)
                plsc.store_scatter(lbuf, [kdest], lv_l)
                plsc.store_scatter(jbuf, [kdest], lv_j)
                plsc.store_scatter(lbuf, [nf + iota], lpad)
                plsc.store_scatter(jbuf, [nf + iota], jpad)

            return (jnp.where(full, fill - FL, fill),
                    cnt + full.astype(jnp.int32))

        fill, cnt = lax.fori_loop(0, CHUNKS, scan_body,
                                  (jnp.int32(0), jnp.int32(0)))

        plsc.store_scatter(lbuf, [fill + iota], lpad)
        plsc.store_scatter(jbuf, [fill + iota], jpad)

        @pl.when(fill > 0)
        def _():
            flush(cnt)

        cntf = cnt + (fill > 0).astype(jnp.int32)

        @pl.when(cntf >= 1)
        def _():
            q = (cntf - 1) & 1
            pltpu.make_async_copy(val.at[jfl.at[q]], valbuf.at[q],
                                  gsem.at[q]).wait()
            pltpu.async_copy(valbuf.at[q], seg.at[lfl.at[q]], ssem.at[q],
                             add=True)
            pltpu.make_async_copy(valbuf.at[q], seg.at[lfl.at[q]],
                                  ssem.at[q]).wait()

        @pl.when(cntf >= 2)
        def _():
            q2 = cntf & 1
            pltpu.make_async_copy(valbuf.at[q2], seg.at[lfl.at[q2]],
                                  ssem.at[q2]).wait()

        plsc.subcore_barrier()
        pltpu.sync_copy(seg.at[pl.ds(start_t, TPT)], out.at[pl.ds(row0, TPT)])
        plsc.subcore_barrier()
        return 0

    lax.fori_loop(0, NPASS, pass_body, 0)


@jax.jit
def _scatter_add(memp, valp, idx):
    mesh = plsc.VectorSubcoreMesh(
        core_axis_name="c", subcore_axis_name="s", num_cores=NC, num_subcores=NT
    )
    return pl.kernel(
        _body,
        out_type=jax.ShapeDtypeStruct((M, DP), jnp.float32),
        mesh=mesh,
        compiler_params=pltpu.CompilerParams(
            needs_layout_passes=False, use_tc_tiling_on_sc=False
        ),
        scratch_types=[
            pltpu.VMEM_SHARED((SEG + L, DP), jnp.float32),
            pltpu.VMEM((IDXSL,), jnp.int32),
            pltpu.VMEM((BUFA,), jnp.int32),
            pltpu.VMEM((BUFA,), jnp.int32),
            pltpu.VMEM((2, FL), jnp.int32),
            pltpu.VMEM((2, FL), jnp.int32),
            pltpu.VMEM((2, FL, DP), jnp.float32),
            pltpu.SemaphoreType.DMA((2,)),
            pltpu.SemaphoreType.DMA((2,)),
        ],
    )(memp, valp, idx)


def kernel(mem, val, idx):
    memp = jnp.pad(mem, ((0, 0), (0, DP - D)))
    valp = jnp.pad(val, ((0, 0), (0, DP - D)))
    outp = _scatter_add(memp, valp, idx)
    return outp[:, :D]
